# Initial kernel scaffold; baseline (speedup 1.0000x reference)
import jax
import jax.numpy as jnp
from jax.experimental import pallas as pl


def kernel(data, atom_table, conv_params, Wc, bc):
    raise NotImplementedError("write your pallas kernel here")


# speedup vs baseline: 17.4944x; 17.4944x over previous
import functools

import jax
import jax.numpy as jnp
from jax import lax
from jax.experimental import pallas as pl
from jax.experimental.pallas import tpu as pltpu
from jax.experimental.pallas import tpu_sc as plsc

B, N, M = 4, 10000, 32
H = 16
C = 32
NF = 16
STEP = 0.2
R = N * M
BR = B * R
NW = 32
ROWBLK = 8
CHUNK = ROWBLK * 128
NCH_W = 40
BR_PAD = NW * NCH_W * CHUNK

BLK_N = 400
BLK_R = BLK_N * M
NBLK = N // BLK_N


def _sc_gather_call(table, idx2d):
    mesh = plsc.VectorSubcoreMesh(core_axis_name="c", subcore_axis_name="s",
                                  num_cores=2, num_subcores=16)

    @functools.partial(
        pl.kernel,
        out_type=jax.ShapeDtypeStruct((BR_PAD // 128, 128, H), jnp.float32),
        mesh=mesh,
        scratch_types=[
            pltpu.VMEM((ROWBLK, 128), jnp.int32),
            pltpu.VMEM((ROWBLK, 128), jnp.int32),
            pltpu.VMEM((ROWBLK, 128, H), jnp.float32),
            pltpu.VMEM((ROWBLK, 128, H), jnp.float32),
            pltpu.SemaphoreType.DMA,
            pltpu.SemaphoreType.DMA,
            pltpu.SemaphoreType.DMA,
            pltpu.SemaphoreType.DMA,
            pltpu.SemaphoreType.DMA,
        ],
        compiler_params=pltpu.CompilerParams(use_tc_tiling_on_sc=False),
    )
    def k(table_hbm, idx_hbm, out_hbm, i0, i1, r0, r1, si0, si1, sg, so0, so1):
        wid = lax.axis_index("s") * 2 + lax.axis_index("c")
        base = wid * NCH_W

        def idx_cp(c, iref, sem):
            return pltpu.make_async_copy(
                idx_hbm.at[pl.ds((base + c) * ROWBLK, ROWBLK)], iref, sem)

        def out_cp(c, rref, sem):
            return pltpu.make_async_copy(
                rref, out_hbm.at[pl.ds((base + c) * ROWBLK, ROWBLK)], sem)

        idx_cp(0, i0, si0).start()

        def body(c2, carry):
            c = c2 * 2
            idx_cp(c, i0, si0).wait()
            idx_cp(c + 1, i1, si1).start()

            @pl.when(c2 > 0)
            def _():
                out_cp(c - 2, r0, so0).wait()
            gd = [pltpu.async_copy(table_hbm.at[i0.at[k]], r0.at[k], sg)
                  for k in range(ROWBLK)]
            for g in gd:
                g.wait()
            out_cp(c, r0, so0).start()

            idx_cp(c + 1, i1, si1).wait()

            @pl.when(c2 < NCH_W // 2 - 1)
            def _():
                idx_cp(c + 2, i0, si0).start()

            @pl.when(c2 > 0)
            def _():
                out_cp(c - 1, r1, so1).wait()
            gd = [pltpu.async_copy(table_hbm.at[i1.at[k]], r1.at[k], sg)
                  for k in range(ROWBLK)]
            for g in gd:
                g.wait()
            out_cp(c + 1, r1, so1).start()
            return carry

        lax.fori_loop(0, NCH_W // 2, body, 0)
        out_cp(NCH_W - 2, r0, so0).wait()
        out_cp(NCH_W - 1, r1, so1).wait()

    return k(table, idx2d)


def _pre_block(emb_r, ag_r, dist_r, w_r, b_r):
    emb = emb_r[0]
    ag = ag_r[...]
    d = dist_r[0]
    w = w_r[...]
    fj = lax.broadcasted_iota(jnp.int32, (NF, BLK_R), 0).astype(jnp.float32) * STEP
    e_t = jnp.exp((d - fj) * (d - fj) * (-1.0 / (STEP * STEP)))
    y3 = lax.dot_general(e_t, w[2 * H:, :], (((0,), (0,)), ((), ())),
                         preferred_element_type=jnp.float32)
    y2 = lax.dot_general(ag, w[H:2 * H, :], (((1,), (0,)), ((), ())),
                         preferred_element_type=jnp.float32)
    y1 = lax.dot_general(emb, w[:H, :], (((1,), (0,)), ((), ())),
                         preferred_element_type=jnp.float32)
    y1b = jnp.broadcast_to(y1[:, None, :], (BLK_N, M, C)).reshape(BLK_R, C)
    return y1b + y2 + y3 + b_r[...]


def _pass1_body(emb_r, ag_r, dist_r, w_r, b_r, out_r):
    @pl.when((pl.program_id(0) == 0) & (pl.program_id(1) == 0))
    def _():
        out_r[...] = jnp.zeros_like(out_r)
    pre = _pre_block(emb_r, ag_r, dist_r, w_r, b_r)
    out_r[0:1, :] += jnp.sum(pre, axis=0, keepdims=True)
    out_r[1:2, :] += jnp.sum(pre * pre, axis=0, keepdims=True)


def _pass2_body(emb_r, ag_r, dist_r, w_r, b_r, al_r, be_r, ns_r, st2_r):
    @pl.when((pl.program_id(0) == 0) & (pl.program_id(1) == 0))
    def _():
        st2_r[...] = jnp.zeros_like(st2_r)
    pre = _pre_block(emb_r, ag_r, dist_r, w_r, b_r)
    y = pre * al_r[...] + be_r[...]
    gate = jax.nn.sigmoid(y[:, :H]) * jnp.maximum(y[:, H:], 0.0)
    ns = jnp.sum(gate.reshape(BLK_N, M, H), axis=1)
    ns_r[0] = ns
    st2_r[0:1, :] += jnp.sum(ns, axis=0, keepdims=True)
    st2_r[1:2, :] += jnp.sum(ns * ns, axis=0, keepdims=True)


def _pass3_body(emb_r, ns_r, a2_r, b2_r, out_r, prot_r):
    new = jnp.maximum(emb_r[0] + ns_r[0] * a2_r[...] + b2_r[...], 0.0)
    out_r[0] = new
    prot_r[0] = jnp.sum(new, axis=0, keepdims=True)


def kernel(data, atom_table, conv_params, Wc, bc):
    f32 = jnp.float32
    dist2 = data[:, :, :M].reshape(B * NBLK, 1, BLK_R).astype(f32)
    idx = data[:, :, M:].astype(jnp.int32)
    idxg = (idx + (jnp.arange(B, dtype=jnp.int32) * N)[:, None, None]
            ).reshape(BR)
    idxp = jnp.concatenate(
        [idxg, jnp.zeros((BR_PAD - BR,), jnp.int32)]).reshape(BR_PAD // 128, 128)

    emb = jnp.broadcast_to(atom_table[None, :, :], (B, N, H)).astype(f32)

    grid = (B, NBLK)
    in_specs = [
        pl.BlockSpec((1, BLK_N, H), lambda b, j: (b, j, 0)),
        pl.BlockSpec((BLK_R, H), lambda b, j: (b * NBLK + j, 0)),
        pl.BlockSpec((1, 1, BLK_R), lambda b, j: (b * NBLK + j, 0, 0)),
        pl.BlockSpec((3 * H, C), lambda b, j: (0, 0)),
        pl.BlockSpec((1, C), lambda b, j: (0, 0)),
    ]

    for (W, b_, g_h, bt_h, g_o, bt_o) in conv_params:
        agp = _sc_gather_call(emb.reshape(B * N, H), idxp)
        agp = agp.reshape(BR_PAD, H)

        st1 = pl.pallas_call(
            _pass1_body,
            grid=grid,
            in_specs=in_specs,
            out_specs=pl.BlockSpec((2, C), lambda b, j: (0, 0)),
            out_shape=jax.ShapeDtypeStruct((2, C), f32),
        )(emb, agp, dist2, W, b_.reshape(1, C))

        cnt1 = float(BR)
        mean1 = st1[0] / cnt1
        var1 = st1[1] / cnt1 - mean1 * mean1
        al = (g_h / jnp.sqrt(var1 + 1e-5)).reshape(1, C)
        be = (bt_h - mean1 * al[0]).reshape(1, C)

        ns, st2 = pl.pallas_call(
            _pass2_body,
            grid=grid,
            in_specs=in_specs + [
                pl.BlockSpec((1, C), lambda b, j: (0, 0)),
                pl.BlockSpec((1, C), lambda b, j: (0, 0)),
            ],
            out_specs=[
                pl.BlockSpec((1, BLK_N, H), lambda b, j: (b, j, 0)),
                pl.BlockSpec((2, H), lambda b, j: (0, 0)),
            ],
            out_shape=[
                jax.ShapeDtypeStruct((B, N, H), f32),
                jax.ShapeDtypeStruct((2, H), f32),
            ],
        )(emb, agp, dist2, W, b_.reshape(1, C), al, be)

        cnt2 = float(B * N)
        mean2 = st2[0] / cnt2
        var2 = st2[1] / cnt2 - mean2 * mean2
        a2 = (g_o / jnp.sqrt(var2 + 1e-5)).reshape(1, H)
        b2 = (bt_o - mean2 * a2[0]).reshape(1, H)

        emb, prot = pl.pallas_call(
            _pass3_body,
            grid=(B,),
            in_specs=[
                pl.BlockSpec((1, N, H), lambda b: (b, 0, 0)),
                pl.BlockSpec((1, N, H), lambda b: (b, 0, 0)),
                pl.BlockSpec((1, H), lambda b: (0, 0)),
                pl.BlockSpec((1, H), lambda b: (0, 0)),
            ],
            out_specs=[
                pl.BlockSpec((1, N, H), lambda b: (b, 0, 0)),
                pl.BlockSpec((1, 1, H), lambda b: (b, 0, 0)),
            ],
            out_shape=[
                jax.ShapeDtypeStruct((B, N, H), f32),
                jax.ShapeDtypeStruct((B, 1, H), f32),
            ],
        )(emb, ns, a2, b2)

    logits = (prot[:, 0, :] / float(N)) @ Wc + bc
    return jax.nn.softmax(logits, axis=-1)
